# baseline (speedup 1.0000x reference)
import dataclasses
import functools

import jax
import jax.numpy as jnp
from jax import lax
from jax.experimental import pallas as pl
from jax.experimental.pallas import tpu as pltpu
from jax.experimental.pallas import tpu_sc as plsc

N = 10000
E = 320000
D = 128
HD = D // 2

NC = 2
NS = 16
NW = NC * NS

NACC = 10496
RPT = NACC // NS
CHUNK = 128
CPT = 80
EPT = CPT * CHUNK
PPT = EPT - E // NW
ZR = 82
NBUF = 4

_MESH = dict(core_axis_name="c", subcore_axis_name="s")

_SC_PARAMS = pltpu.CompilerParams()
if "needs_layout_passes" in pltpu.CompilerParams.__dataclass_fields__:
    _SC_PARAMS = dataclasses.replace(_SC_PARAMS, needs_layout_passes=False)
if "use_tc_tiling_on_sc" in pltpu.CompilerParams.__dataclass_fields__:
    _SC_PARAMS = dataclasses.replace(_SC_PARAMS, use_tc_tiling_on_sc=False)


def _rsqrt_inplace(nbuf, rows):
    @pl.loop(0, rows, step=16)
    def _(q):
        x = jnp.maximum(nbuf[pl.ds(q, 16)], 1.0)
        xi = plsc.bitcast(x, jnp.int32)
        yi = jnp.full((16,), 0x5F3759DF, jnp.int32) - lax.shift_right_logical(
            xi, jnp.full((16,), 1, jnp.int32))
        y = plsc.bitcast(yi, jnp.float32)
        for _ in range(3):
            y = y * (1.5 - 0.5 * x * y * y)
        nbuf[pl.ds(q, 16)] = y


def _deg_norms(srcp, dstp):
    mesh = plsc.VectorSubcoreMesh(**_MESH)

    @functools.partial(
        pl.kernel,
        out_type=(jax.ShapeDtypeStruct((NACC,), jnp.float32),
                  jax.ShapeDtypeStruct((NACC,), jnp.float32)),
        mesh=mesh,
        scratch_types=[
            pltpu.VMEM((CPT, CHUNK), jnp.int32),
            pltpu.VMEM((CHUNK,), jnp.float32),
            pltpu.VMEM((RPT,), jnp.float32),
            pltpu.VMEM_SHARED((NACC,), jnp.float32),
        ],
        compiler_params=_SC_PARAMS,
    )
    def k(srcp_hbm, dstp_hbm, ns_hbm, nd_hbm, idxb, ones, nbuf, deg):
        c = lax.axis_index("c")
        s = lax.axis_index("s")

        @pl.loop(0, CHUNK, step=16)
        def _(q):
            ones[pl.ds(q, 16)] = jnp.ones((16,), jnp.float32)
        @pl.loop(0, RPT, step=16)
        def _(q):
            nbuf[pl.ds(q, 16)] = jnp.zeros((16,), jnp.float32)
        base = s * RPT
        pltpu.sync_copy(nbuf, deg.at[pl.ds(base, RPT)])
        plsc.subcore_barrier()
        for half in range(2):
            w = half * NS + s
            @pl.when(c == 0)
            def _():
                pltpu.sync_copy(srcp_hbm.at[w], idxb)
            @pl.when(c == 1)
            def _():
                pltpu.sync_copy(dstp_hbm.at[w], idxb)
            @pl.loop(0, CPT)
            def _(j):
                pltpu.sync_copy(ones, deg.at[idxb.at[j]], add=True)
        plsc.subcore_barrier()
        pltpu.sync_copy(deg.at[pl.ds(base, RPT)], nbuf)
        _rsqrt_inplace(nbuf, RPT)
        @pl.when(c == 0)
        def _():
            pltpu.sync_copy(nbuf, ns_hbm.at[pl.ds(base, RPT)])
        @pl.when(c == 1)
        def _():
            pltpu.sync_copy(nbuf, nd_hbm.at[pl.ds(base, RPT)])

    return k(srcp, dstp)


def _make_spmm():
    mesh = plsc.VectorSubcoreMesh(**_MESH)

    @functools.partial(
        pl.kernel,
        out_type=jax.ShapeDtypeStruct((2, NC, NACC, HD), jnp.float32),
        mesh=mesh,
        scratch_types=[
            pltpu.VMEM((CPT, CHUNK), jnp.int32),
            pltpu.VMEM((CPT, CHUNK), jnp.int32),
        ] + [pltpu.VMEM((CHUNK, HD), jnp.float32) for _ in range(NBUF)] + [
            pltpu.VMEM((ZR, HD), jnp.float32),
            pltpu.VMEM_SHARED((NACC, HD), jnp.float32),
        ] + [pltpu.SemaphoreType.DMA for _ in range(NBUF)],
        compiler_params=_SC_PARAMS,
        name="gcn_spmm",
    )
    def k(hwa_hbm, hwb_hbm, srcp_hbm, dstp_hbm, out_hbm, sidx, didx, *rest):
        bufs = rest[:NBUF]
        zb = rest[NBUF]
        acc = rest[NBUF + 1]
        sems = rest[NBUF + 2:]
        c = lax.axis_index("c")
        s = lax.axis_index("s")
        w = c * NS + s
        base = s * RPT

        @pl.loop(0, ZR)
        def _(r):
            @pl.loop(0, HD, step=16)
            def _(q):
                zb[r, pl.ds(q, 16)] = jnp.zeros((16,), jnp.float32)

        pltpu.sync_copy(srcp_hbm.at[w], sidx)
        pltpu.sync_copy(dstp_hbm.at[w], didx)

        for hf, src_hbm in enumerate((hwa_hbm, hwb_hbm)):
            for t in range(RPT // ZR):
                pltpu.sync_copy(zb, acc.at[pl.ds(base + t * ZR, ZR)])
            plsc.subcore_barrier()

            for b in range(NBUF):
                pltpu.async_copy(src_hbm.at[sidx.at[b]], bufs[b], sems[b])

            @pl.loop(0, CPT, step=NBUF)
            def _(j):
                for b in range(NBUF):
                    pltpu.make_async_copy(
                        src_hbm.at[sidx.at[j + b]], bufs[b], sems[b]).wait()
                    pltpu.sync_copy(bufs[b], acc.at[didx.at[j + b]], add=True)
                    @pl.when(j + b + NBUF < CPT)
                    def _():
                        pltpu.async_copy(
                            src_hbm.at[sidx.at[j + b + NBUF]], bufs[b], sems[b])

            plsc.subcore_barrier()
            pltpu.sync_copy(acc.at[pl.ds(base, RPT)],
                            out_hbm.at[hf, c, pl.ds(base, RPT)])

    return k


_spmm = _make_spmm()


def _tc_stage_a(x, W, ns_col):
    def body(x_ref, w_ref, ns_ref, oa_ref, ob_ref):
        xw = jnp.dot(x_ref[...], w_ref[...], preferred_element_type=jnp.float32)
        hw = xw * ns_ref[...]
        oa_ref[pl.ds(0, N), :] = hw[:, :HD]
        ob_ref[pl.ds(0, N), :] = hw[:, HD:]
        zpad = jnp.zeros((NACC - N, HD), jnp.float32)
        oa_ref[pl.ds(N, NACC - N), :] = zpad
        ob_ref[pl.ds(N, NACC - N), :] = zpad

    return pl.pallas_call(
        body, out_shape=(jax.ShapeDtypeStruct((NACC, HD), jnp.float32),
                         jax.ShapeDtypeStruct((NACC, HD), jnp.float32)),
    )(x, W, ns_col)


def _agg_sum(a_ref):
    agg_a = a_ref[0, 0, pl.ds(0, N), :] + a_ref[0, 1, pl.ds(0, N), :]
    agg_b = a_ref[1, 0, pl.ds(0, N), :] + a_ref[1, 1, pl.ds(0, N), :]
    return jnp.concatenate([agg_a, agg_b], axis=1)


def _tc_stage_mid(agg, nd_col, b_in, W, ns_col):
    def body(a_ref, nd_ref, b_ref, w_ref, ns_ref, oa_ref, ob_ref):
        h = jnp.maximum(_agg_sum(a_ref) * nd_ref[...] + b_ref[...], 0.0)
        hw = jnp.dot(h, w_ref[...], preferred_element_type=jnp.float32)
        hw = hw * ns_ref[...]
        oa_ref[pl.ds(0, N), :] = hw[:, :HD]
        ob_ref[pl.ds(0, N), :] = hw[:, HD:]
        zpad = jnp.zeros((NACC - N, HD), jnp.float32)
        oa_ref[pl.ds(N, NACC - N), :] = zpad
        ob_ref[pl.ds(N, NACC - N), :] = zpad

    return pl.pallas_call(
        body, out_shape=(jax.ShapeDtypeStruct((NACC, HD), jnp.float32),
                         jax.ShapeDtypeStruct((NACC, HD), jnp.float32)),
    )(agg, nd_col, b_in, W, ns_col)


def _tc_stage_out(agg, nd_col, b_in, W, b_out):
    def body(a_ref, nd_ref, b_ref, w_ref, bo_ref, o_ref):
        h = jnp.maximum(_agg_sum(a_ref) * nd_ref[...] + b_ref[...], 0.0)
        hw = jnp.dot(h, w_ref[...], preferred_element_type=jnp.float32)
        o_ref[...] = hw + bo_ref[...]

    return pl.pallas_call(
        body, out_shape=jax.ShapeDtypeStruct((N, D), jnp.float32),
    )(agg, nd_col, b_in, W, b_out)


def kernel(features, edge_index, W1, b1, W2, b2, W3, b3):
    src = edge_index[0]
    dst = edge_index[1]
    padv = (N + jnp.arange(NW * PPT, dtype=jnp.int32) % (NACC - N)).reshape(
        NW, PPT)
    srcp = jnp.concatenate(
        [src.reshape(NW, E // NW), padv], axis=1).reshape(NW, CPT, CHUNK)
    dstp = jnp.concatenate(
        [dst.reshape(NW, E // NW), padv], axis=1).reshape(NW, CPT, CHUNK)

    ns_vec, nd_vec = _deg_norms(srcp, dstp)
    ns_col = ns_vec[:N].reshape(N, 1)
    nd_col = nd_vec[:N].reshape(N, 1)

    hw1a, hw1b = _tc_stage_a(features, W1, ns_col)
    agg1 = _spmm(hw1a, hw1b, srcp, dstp)
    hw2a, hw2b = _tc_stage_mid(agg1, nd_col, b1.reshape(1, D), W2, ns_col)
    agg2 = _spmm(hw2a, hw2b, srcp, dstp)
    return _tc_stage_out(agg2, nd_col, b2.reshape(1, D), W3, b3.reshape(1, D))


# speedup vs baseline: 1.0001x; 1.0001x over previous
import dataclasses
import functools

import jax
import jax.numpy as jnp
from jax import lax
from jax.experimental import pallas as pl
from jax.experimental.pallas import tpu as pltpu
from jax.experimental.pallas import tpu_sc as plsc

N = 10000
E = 320000
D = 128
HD = D // 2

NC = 2
NS = 16
NW = NC * NS

NACC = 10496
RPT = NACC // NS
CHUNK = 128
CPT = 80
EPT = CPT * CHUNK
PPT = EPT - E // NW
ZR = 82
NBUF = 4

_MESH = dict(core_axis_name="c", subcore_axis_name="s")

_SC_PARAMS = pltpu.CompilerParams()
if "needs_layout_passes" in pltpu.CompilerParams.__dataclass_fields__:
    _SC_PARAMS = dataclasses.replace(_SC_PARAMS, needs_layout_passes=False)
if "use_tc_tiling_on_sc" in pltpu.CompilerParams.__dataclass_fields__:
    _SC_PARAMS = dataclasses.replace(_SC_PARAMS, use_tc_tiling_on_sc=False)


def _rsqrt_inplace(nbuf, rows):
    @pl.loop(0, rows, step=16)
    def _(q):
        x = jnp.maximum(nbuf[pl.ds(q, 16)], 1.0)
        xi = plsc.bitcast(x, jnp.int32)
        yi = jnp.full((16,), 0x5F3759DF, jnp.int32) - lax.shift_right_logical(
            xi, jnp.full((16,), 1, jnp.int32))
        y = plsc.bitcast(yi, jnp.float32)
        for _ in range(3):
            y = y * (1.5 - 0.5 * x * y * y)
        nbuf[pl.ds(q, 16)] = y


def _deg_norms(srcp, dstp):
    mesh = plsc.VectorSubcoreMesh(**_MESH)

    @functools.partial(
        pl.kernel,
        out_type=(jax.ShapeDtypeStruct((NACC,), jnp.float32),
                  jax.ShapeDtypeStruct((NACC,), jnp.float32)),
        mesh=mesh,
        scratch_types=[
            pltpu.VMEM((CPT, CHUNK), jnp.int32),
            pltpu.VMEM((CHUNK,), jnp.float32),
            pltpu.VMEM((RPT,), jnp.float32),
            pltpu.VMEM_SHARED((NACC,), jnp.float32),
        ],
        compiler_params=_SC_PARAMS,
    )
    def k(srcp_hbm, dstp_hbm, ns_hbm, nd_hbm, idxb, ones, nbuf, deg):
        c = lax.axis_index("c")
        s = lax.axis_index("s")

        @pl.loop(0, CHUNK, step=16)
        def _(q):
            ones[pl.ds(q, 16)] = jnp.ones((16,), jnp.float32)
        @pl.loop(0, RPT, step=16)
        def _(q):
            nbuf[pl.ds(q, 16)] = jnp.zeros((16,), jnp.float32)
        base = s * RPT
        pltpu.sync_copy(nbuf, deg.at[pl.ds(base, RPT)])
        plsc.subcore_barrier()
        for half in range(2):
            w = half * NS + s
            @pl.when(c == 0)
            def _():
                pltpu.sync_copy(srcp_hbm.at[w], idxb)
            @pl.when(c == 1)
            def _():
                pltpu.sync_copy(dstp_hbm.at[w], idxb)
            @pl.loop(0, CPT)
            def _(j):
                pltpu.sync_copy(ones, deg.at[idxb.at[j]], add=True)
        plsc.subcore_barrier()
        pltpu.sync_copy(deg.at[pl.ds(base, RPT)], nbuf)
        _rsqrt_inplace(nbuf, RPT)
        @pl.when(c == 0)
        def _():
            pltpu.sync_copy(nbuf, ns_hbm.at[pl.ds(base, RPT)])
        @pl.when(c == 1)
        def _():
            pltpu.sync_copy(nbuf, nd_hbm.at[pl.ds(base, RPT)])

    return k(srcp, dstp)


def _make_spmm():
    mesh = plsc.VectorSubcoreMesh(**_MESH)

    @functools.partial(
        pl.kernel,
        out_type=jax.ShapeDtypeStruct((2, NC, NACC, HD), jnp.float32),
        mesh=mesh,
        scratch_types=[
            pltpu.VMEM((CPT, CHUNK), jnp.int32),
            pltpu.VMEM((CPT, CHUNK), jnp.int32),
        ] + [pltpu.VMEM((CHUNK, HD), jnp.float32) for _ in range(NBUF)] + [
            pltpu.VMEM((ZR, HD), jnp.float32),
            pltpu.VMEM_SHARED((NACC, HD), jnp.float32),
        ] + [pltpu.SemaphoreType.DMA for _ in range(NBUF)],
        compiler_params=_SC_PARAMS,
        name="gcn_spmm",
    )
    def k(hwa_hbm, hwb_hbm, srcp_hbm, dstp_hbm, out_hbm, sidx, didx, *rest):
        bufs = rest[:NBUF]
        zb = rest[NBUF]
        acc = rest[NBUF + 1]
        sems = rest[NBUF + 2:]
        c = lax.axis_index("c")
        s = lax.axis_index("s")
        w = c * NS + s
        base = s * RPT

        @pl.loop(0, ZR)
        def _(r):
            @pl.loop(0, HD, step=16)
            def _(q):
                zb[r, pl.ds(q, 16)] = jnp.zeros((16,), jnp.float32)

        pltpu.sync_copy(srcp_hbm.at[w], sidx)
        pltpu.sync_copy(dstp_hbm.at[w], didx)

        for hf, src_hbm in enumerate((hwa_hbm, hwb_hbm)):
            for t in range(RPT // ZR):
                pltpu.sync_copy(zb, acc.at[pl.ds(base + t * ZR, ZR)])
            plsc.subcore_barrier()

            for b in range(NBUF):
                pltpu.async_copy(src_hbm.at[sidx.at[b]], bufs[b], sems[b])

            @pl.loop(0, CPT, step=NBUF)
            def _(j):
                for b in range(NBUF):
                    pltpu.make_async_copy(
                        src_hbm.at[sidx.at[j + b]], bufs[b], sems[b]).wait()
                    pltpu.sync_copy(bufs[b], acc.at[didx.at[j + b]], add=True)
                    @pl.when(j + b + NBUF < CPT)
                    def _():
                        pltpu.async_copy(
                            src_hbm.at[sidx.at[j + b + NBUF]], bufs[b], sems[b])

            plsc.subcore_barrier()
            pltpu.sync_copy(acc.at[pl.ds(base, RPT)],
                            out_hbm.at[hf, c, pl.ds(base, RPT)])

    return k


_spmm = _make_spmm()


def _tc_matmul(x, W):
    def body(x_ref, w_ref, o_ref):
        o_ref[...] = jnp.dot(x_ref[...], w_ref[...],
                             preferred_element_type=jnp.float32)

    return pl.pallas_call(
        body, out_shape=jax.ShapeDtypeStruct((N, D), jnp.float32),
    )(x, W)


def _tc_scale_split(xw, ns_col):
    def body(x_ref, ns_ref, oa_ref, ob_ref):
        hw = x_ref[...] * ns_ref[...]
        oa_ref[pl.ds(0, N), :] = hw[:, :HD]
        ob_ref[pl.ds(0, N), :] = hw[:, HD:]
        zpad = jnp.zeros((NACC - N, HD), jnp.float32)
        oa_ref[pl.ds(N, NACC - N), :] = zpad
        ob_ref[pl.ds(N, NACC - N), :] = zpad

    return pl.pallas_call(
        body, out_shape=(jax.ShapeDtypeStruct((NACC, HD), jnp.float32),
                         jax.ShapeDtypeStruct((NACC, HD), jnp.float32)),
    )(xw, ns_col)


def _agg_sum(a_ref):
    agg_a = a_ref[0, 0, pl.ds(0, N), :] + a_ref[0, 1, pl.ds(0, N), :]
    agg_b = a_ref[1, 0, pl.ds(0, N), :] + a_ref[1, 1, pl.ds(0, N), :]
    return jnp.concatenate([agg_a, agg_b], axis=1)


def _tc_stage_mid(agg, nd_col, b_in, W, ns_col):
    def body(a_ref, nd_ref, b_ref, w_ref, ns_ref, oa_ref, ob_ref):
        h = jnp.maximum(_agg_sum(a_ref) * nd_ref[...] + b_ref[...], 0.0)
        hw = jnp.dot(h, w_ref[...], preferred_element_type=jnp.float32)
        hw = hw * ns_ref[...]
        oa_ref[pl.ds(0, N), :] = hw[:, :HD]
        ob_ref[pl.ds(0, N), :] = hw[:, HD:]
        zpad = jnp.zeros((NACC - N, HD), jnp.float32)
        oa_ref[pl.ds(N, NACC - N), :] = zpad
        ob_ref[pl.ds(N, NACC - N), :] = zpad

    return pl.pallas_call(
        body, out_shape=(jax.ShapeDtypeStruct((NACC, HD), jnp.float32),
                         jax.ShapeDtypeStruct((NACC, HD), jnp.float32)),
    )(agg, nd_col, b_in, W, ns_col)


def _tc_stage_out(agg, nd_col, b_in, W, b_out):
    def body(a_ref, nd_ref, b_ref, w_ref, bo_ref, o_ref):
        h = jnp.maximum(_agg_sum(a_ref) * nd_ref[...] + b_ref[...], 0.0)
        hw = jnp.dot(h, w_ref[...], preferred_element_type=jnp.float32)
        o_ref[...] = hw + bo_ref[...]

    return pl.pallas_call(
        body, out_shape=jax.ShapeDtypeStruct((N, D), jnp.float32),
    )(agg, nd_col, b_in, W, b_out)


def kernel(features, edge_index, W1, b1, W2, b2, W3, b3):
    src = edge_index[0]
    dst = edge_index[1]
    padv = (N + jnp.arange(NW * PPT, dtype=jnp.int32) % (NACC - N)).reshape(
        NW, PPT)
    srcp = jnp.concatenate(
        [src.reshape(NW, E // NW), padv], axis=1).reshape(NW, CPT, CHUNK)
    dstp = jnp.concatenate(
        [dst.reshape(NW, E // NW), padv], axis=1).reshape(NW, CPT, CHUNK)

    ns_vec, nd_vec = _deg_norms(srcp, dstp)
    xw = _tc_matmul(features, W1)
    ns_col = ns_vec[:N].reshape(N, 1)
    nd_col = nd_vec[:N].reshape(N, 1)

    hw1a, hw1b = _tc_scale_split(xw, ns_col)
    agg1 = _spmm(hw1a, hw1b, srcp, dstp)
    hw2a, hw2b = _tc_stage_mid(agg1, nd_col, b1.reshape(1, D), W2, ns_col)
    agg2 = _spmm(hw2a, hw2b, srcp, dstp)
    return _tc_stage_out(agg2, nd_col, b2.reshape(1, D), W3, b3.reshape(1, D))
